# baseline (speedup 1.0000x reference)
import jax
import jax.numpy as jnp
from jax.experimental import pallas as pl
from jax.experimental.pallas import tpu as pltpu

_E = 8
_TOPK = 2
_CHUNK = 1024
_NBUF = 4


def _router_body(x_hbm, w_ref, b_ref, logits_ref, vals_ref, idx_ref, xbuf, sems):
    i = pl.program_id(0)
    n = pl.num_programs(0)
    slot = jax.lax.rem(i, _NBUF)

    @pl.when(i == 0)
    def _prologue():
        for k in range(_NBUF):
            pltpu.make_async_copy(
                x_hbm.at[pl.ds(k * _CHUNK, _CHUNK), :], xbuf.at[k], sems.at[k]
            ).start()

    pltpu.make_async_copy(
        x_hbm.at[pl.ds(i * _CHUNK, _CHUNK), :], xbuf.at[slot], sems.at[slot]
    ).wait()

    x = xbuf[slot]
    logits = jax.lax.dot_general(
        x, w_ref[...], (((1,), (1,)), ((), ())), preferred_element_type=jnp.float32
    ) + b_ref[...]
    logits_ref[...] = logits

    m1 = jnp.max(logits, axis=-1, keepdims=True)
    i1 = jnp.argmax(logits, axis=-1)
    iota = jax.lax.broadcasted_iota(jnp.int32, logits.shape, 1)
    masked = jnp.where(iota == i1[:, None], -jnp.inf, logits)
    m2 = jnp.max(masked, axis=-1, keepdims=True)
    i2 = jnp.argmax(masked, axis=-1)

    w1 = 1.0 / (1.0 + jnp.exp(m2 - m1))
    vals_ref[...] = jnp.concatenate([w1, 1.0 - w1], axis=1)
    idx_ref[...] = jnp.concatenate([i1[:, None], i2[:, None]], axis=1)

    nxt = i + _NBUF

    @pl.when(nxt < n)
    def _prefetch():
        pltpu.make_async_copy(
            x_hbm.at[pl.ds(nxt * _CHUNK, _CHUNK), :], xbuf.at[slot], sems.at[slot]
        ).start()


def kernel(hidden_states, W, b):
    orig_shape = hidden_states.shape
    x = hidden_states.reshape(-1, orig_shape[-1])
    n_tokens, hidden = x.shape
    grid = (n_tokens // _CHUNK,)

    logits, vals, idx = pl.pallas_call(
        _router_body,
        grid=grid,
        in_specs=[
            pl.BlockSpec(memory_space=pl.ANY),
            pl.BlockSpec((_E, hidden), lambda i: (0, 0)),
            pl.BlockSpec((1, _E), lambda i: (0, 0)),
        ],
        out_specs=[
            pl.BlockSpec((_CHUNK, _E), lambda i: (i, 0)),
            pl.BlockSpec((_CHUNK, _TOPK), lambda i: (i, 0)),
            pl.BlockSpec((_CHUNK, _TOPK), lambda i: (i, 0)),
        ],
        out_shape=[
            jax.ShapeDtypeStruct((n_tokens, _E), jnp.float32),
            jax.ShapeDtypeStruct((n_tokens, _TOPK), jnp.float32),
            jax.ShapeDtypeStruct((n_tokens, _TOPK), jnp.int32),
        ],
        scratch_shapes=[
            pltpu.VMEM((_NBUF, _CHUNK, hidden), jnp.float32),
            pltpu.SemaphoreType.DMA((_NBUF,)),
        ],
        compiler_params=pltpu.CompilerParams(
            dimension_semantics=("arbitrary",),
        ),
    )(x, W, b.reshape(1, _E))

    return (logits, vals, idx)
